# baseline (speedup 1.0000x reference)
import jax
import jax.numpy as jnp
from jax import lax
from jax.experimental import pallas as pl
from jax.experimental.pallas import tpu as pltpu
from jax.experimental.pallas import tpu_sc as plsc

_N_CLASSES = 1000
_D = 768
_CONTEXT_LENGTH = 77
_N_CTX = 17
_CTX_LEN = _N_CTX - 1
_SUFFIX_LEN = _CONTEXT_LENGTH - _N_CTX

_NCHUNK = _N_CLASSES // 8
_NJC = _D // 128
_BLK = 6 * 8
_PBLK = 2 * _BLK
_NBUF = 6
_PREP = 4
_BIG = 1 << 30


def _sc_body(ids_hbm, t2_hbm, pfx2_hbm, ctx2_hbm,
             out2_hbm, eos_hbm,
             headk, headtmp, gbufs_t, ibs_t, idsv, idxcb, eosv,
             hgsem, hwsem, gsems_t, wsems_t):
    core = lax.axis_index("c")
    sub = lax.axis_index("s")
    wid = sub * 2 + core
    nseg = jnp.where(wid < 30, 2, 1)

    gbufs = tuple(gbufs_t)
    ibs = tuple(ibs_t)
    gsems = tuple(gsems_t)
    wsems = tuple(wsems_t)

    lanes = lax.iota(jnp.int32, 16)
    lane0 = lanes == 0
    lane_s = lanes & 7
    lane_jc8 = (lanes >> 3) * 8

    def pair_of(seg):
        return jnp.where((seg == 0) & (wid < 30), wid, wid + 30)

    for seg in range(2):
        @pl.when(seg < nseg)
        def _():
            pltpu.sync_copy(ids_hbm.at[pl.ds(16 * pair_of(seg), 16)],
                            idsv.at[pl.ds(16 * seg, 16)])

    @pl.when(wid >= 30)
    def _():
        pltpu.sync_copy(ids_hbm.at[pl.ds(992, 8)], idsv.at[pl.ds(16, 8)])

    nu = _SUFFIX_LEN * nseg

    def build_idx(u, b):
        j = u // nseg
        seg = u % nseg
        jv = jnp.full((16,), j, jnp.int32)
        for half in range(2):
            rv = plsc.load_gather(idsv, [16 * seg + 8 * half + lane_s, jv])
            hi = (rv >> 3) * _BLK + (rv & 7)
            for kk in range(3):
                ibs[b][pl.ds(_BLK * half + 16 * kk, 16)] = (
                    hi + 16 * kk + lane_jc8)

    def gather_desc(b):
        return pltpu.make_async_copy(t2_hbm.at[ibs[b]], gbufs[b], gsems[b])

    def write_desc(u, b):
        j = u // nseg
        seg = u % nseg
        return pltpu.make_async_copy(
            gbufs[b],
            out2_hbm.at[pl.ds(
                _BLK * ((_N_CTX + j) * _NCHUNK + 2 * pair_of(seg)), _PBLK)],
            wsems[b])

    for b in range(_PREP):
        build_idx(b, b)
        gather_desc(b).start()

    for t in range(_N_CTX):
        q = t - 1
        for k in range(3):
            if t == 0:
                vec = 2 * k + (lanes >> 3)
            else:
                vec = (q >> 3) * _BLK + 16 * k + lane_jc8 + (q & 7)
            idxcb[pl.ds(_BLK * t + 16 * k, 16)] = vec

    def stage_head(t):
        src = pfx2_hbm if t == 0 else ctx2_hbm
        pltpu.async_copy(src.at[idxcb.at[pl.ds(_BLK * t, _BLK)]],
                         headtmp, hgsem).wait()
        pltpu.sync_copy(headtmp, headk.at[pl.ds(_PBLK * t, _BLK)])
        pltpu.sync_copy(headtmp, headk.at[pl.ds(_PBLK * t + _BLK, _BLK)])

    for t in range(16):
        @pl.when(sub == t)
        def _():
            stage_head(t)

    @pl.when(sub == 0)
    def _():
        stage_head(16)

    plsc.subcore_barrier()

    def head_write_desc(t, seg):
        return pltpu.make_async_copy(
            headk.at[pl.ds(_PBLK * t, _PBLK)],
            out2_hbm.at[pl.ds(_BLK * (t * _NCHUNK + 2 * pair_of(seg)),
                              _PBLK)],
            hwsem)

    def head_write124_desc(t):
        return pltpu.make_async_copy(
            headk.at[pl.ds(_PBLK * t, _BLK)],
            out2_hbm.at[pl.ds(_BLK * (t * _NCHUNK + 124), _BLK)], hwsem)

    for t in range(_N_CTX):
        for seg in range(2):
            @pl.when(seg < nseg)
            def _():
                head_write_desc(t, seg).start()

    @pl.when(wid == 31)
    def _():
        for t in range(_N_CTX):
            head_write124_desc(t).start()

    def eos_body(cls, carry):
        gm = jnp.int32(-1)
        args = jnp.int32(_BIG)
        for off in (0, 16, 32, 44):
            ch = idsv[cls, pl.ds(off, 16)]
            gm = jnp.maximum(gm, jnp.max(ch))
        for off in (0, 16, 32, 44):
            ch = idsv[cls, pl.ds(off, 16)]
            cand = jnp.where(ch == gm, lanes + off, _BIG)
            args = jnp.minimum(args, jnp.min(cand))
        plsc.store_scatter(eosv, [jnp.full((16,), cls, jnp.int32)],
                           jnp.full((16,), args + _N_CTX, jnp.int32),
                           mask=lane0)
        return carry

    lax.fori_loop(0, nseg * 16, eos_body, 0)
    for seg in range(2):
        @pl.when(seg < nseg)
        def _():
            pltpu.sync_copy(eosv.at[pl.ds(16 * seg, 16)],
                            eos_hbm.at[pl.ds(16 * pair_of(seg), 16)])

    @pl.when(wid == 30)
    def _():
        lax.fori_loop(16, 24, eos_body, 0)
        pltpu.sync_copy(eosv.at[pl.ds(16, 8)], eos_hbm.at[pl.ds(992, 8)])

    def ring_iter(u, b):
        @pl.when(u + _PREP < nu)
        def _():
            @pl.when(u >= _NBUF - _PREP)
            def _():
                write_desc(u + _PREP - _NBUF, (b + _PREP) % _NBUF).wait()
            build_idx(u + _PREP, (b + _PREP) % _NBUF)
            gather_desc((b + _PREP) % _NBUF).start()

        gather_desc(b).wait()
        write_desc(u, b).start()

    def body(i, carry):
        for b in range(_NBUF):
            ring_iter(_NBUF * i + b, b)
        return carry

    lax.fori_loop(0, nu // _NBUF, body, 0)

    for b in range(_NBUF):
        write_desc(nu - _NBUF + b, b).wait()

    @pl.when(wid >= 30)
    def _():
        mbase = 15 * (wid - 30)

        def e_build(m, b):
            for half in range(2):
                jv = jnp.full((16,), 2 * (mbase + m) + half, jnp.int32)
                rv = plsc.load_gather(idsv, [16 + lane_s, jv])
                hi = (rv >> 3) * _BLK + (rv & 7)
                for kk in range(3):
                    ibs[b][pl.ds(_BLK * half + 16 * kk, 16)] = (
                        hi + 16 * kk + lane_jc8)

        def e_gather(b):
            return pltpu.make_async_copy(t2_hbm.at[ibs[b]], gbufs[b],
                                         gsems[b])

        def e_write(m, b, half):
            t = _N_CTX + 2 * (mbase + m) + half
            return pltpu.make_async_copy(
                gbufs[b].at[pl.ds(_BLK * half, _BLK)],
                out2_hbm.at[pl.ds(_BLK * (t * _NCHUNK + 124), _BLK)],
                wsems[b])

        e_build(0, 0)
        e_gather(0).start()
        for m in range(15):
            b = m % 2
            if m + 1 < 15:
                if m >= 1:
                    e_write(m - 1, 1 - b, 0).wait()
                    e_write(m - 1, 1 - b, 1).wait()
                e_build(m + 1, 1 - b)
                e_gather(1 - b).start()
            e_gather(b).wait()
            e_write(m, b, 0).start()
            e_write(m, b, 1).start()
        e_write(13, 1, 0).wait()
        e_write(13, 1, 1).wait()
        e_write(14, 0, 0).wait()
        e_write(14, 0, 1).wait()

    for t in range(_N_CTX):
        for seg in range(2):
            @pl.when(seg < nseg)
            def _():
                head_write_desc(t, seg).wait()

    @pl.when(wid == 31)
    def _():
        for t in range(_N_CTX):
            head_write124_desc(t).wait()


@jax.jit
def kernel(token_ids, table, token_prefix, ctx_embedding):
    t2 = (table.reshape(49408 // 8, 8, _NJC, 128)
          .transpose(0, 2, 1, 3).reshape(49408 // 8 * _NJC * 8, 128))
    pfx2 = token_prefix.reshape(_NJC, 128)
    ctx2 = (ctx_embedding.reshape(2, 8, _NJC, 128)
            .transpose(0, 2, 1, 3).reshape(2 * _NJC * 8, 128))

    mesh = plsc.VectorSubcoreMesh(core_axis_name="c", subcore_axis_name="s")
    run = pl.kernel(
        _sc_body,
        out_type=(
            jax.ShapeDtypeStruct((_CONTEXT_LENGTH * _NCHUNK * _BLK, 128),
                                 jnp.float32),
            jax.ShapeDtypeStruct((_N_CLASSES,), jnp.int32),
        ),
        mesh=mesh,
        compiler_params=pltpu.CompilerParams(
            use_tc_tiling_on_sc=False, needs_layout_passes=False),
        scratch_types=[
            pltpu.VMEM_SHARED((_N_CTX * _PBLK, 128), jnp.float32),
            pltpu.VMEM((_BLK, 128), jnp.float32),
            [pltpu.VMEM((_PBLK, 128), jnp.float32)] * _NBUF,
            [pltpu.VMEM((_PBLK,), jnp.int32)] * _NBUF,
            pltpu.VMEM((32, _SUFFIX_LEN), jnp.int32),
            pltpu.VMEM((_N_CTX * _BLK,), jnp.int32),
            pltpu.VMEM((32,), jnp.int32),
            pltpu.SemaphoreType.DMA,
            pltpu.SemaphoreType.DMA,
            [pltpu.SemaphoreType.DMA] * _NBUF,
            [pltpu.SemaphoreType.DMA] * _NBUF,
        ],
    )
    out2, eos = run(token_ids, t2, pfx2, ctx2)
    embeddings = (out2.reshape(_CONTEXT_LENGTH, _NCHUNK, _NJC, 8, 128)
                  .transpose(1, 3, 0, 2, 4)
                  .reshape(_N_CLASSES, _CONTEXT_LENGTH, _D))
    return embeddings, eos
